# baseline (speedup 1.0000x reference)
import dataclasses
import functools

import jax
import jax.numpy as jnp
from jax import lax
from jax.experimental import pallas as pl
from jax.experimental.pallas import tpu as pltpu
from jax.experimental.pallas import tpu_sc as plsc

N = 10000
E = 320000
HP = 256
QN = 80
BM = 1000
NT = 16
CHUNK = 128
NGRP = 4
GCHUNK = 40
NCHUNK = NGRP * GCHUNK
EPT_PAD = NCHUNK * CHUNK
ZROWS = 632
NROWS_PAD = NT * ZROWS
OPT = 624


def _sc_compiler_params():
    cp = pltpu.CompilerParams()
    if "needs_layout_passes" in pltpu.CompilerParams.__dataclass_fields__:
        cp = dataclasses.replace(cp, needs_layout_passes=False)
    return cp


def _prep_body(x_ref, w_ref, cnt_ref, u_ref, hw_ref):
    hw = jnp.dot(x_ref[...], w_ref[...], preferred_element_type=jnp.float32)
    dinv = lax.rsqrt(jnp.maximum(cnt_ref[...] + 1.0, 1.0))
    u_ref[0] = hw * dinv
    hw_ref[...] = hw


def _prep(x, w1p, cnt_col):
    return pl.pallas_call(
        _prep_body,
        grid=(N // BM, 2),
        in_specs=[
            pl.BlockSpec((BM, 128), lambda i, j: (i, 0)),
            pl.BlockSpec((128, 128), lambda i, j: (0, j)),
            pl.BlockSpec((BM, 1), lambda i, j: (i, 0)),
        ],
        out_specs=[
            pl.BlockSpec((1, BM, 128), lambda i, j: (j, i, 0)),
            pl.BlockSpec((BM, 128), lambda i, j: (i, j)),
        ],
        out_shape=[
            jax.ShapeDtypeStruct((2, N, 128), jnp.float32),
            jax.ShapeDtypeStruct((N, HP), jnp.float32),
        ],
    )(x, w1p, cnt_col)


def _mid_body(part_ref, hw_ref, cnt_ref, b_ref, w2_ref, u2_ref, hw2_ref):
    dinv = lax.rsqrt(jnp.maximum(cnt_ref[...] + 1.0, 1.0))
    agg = jnp.concatenate([part_ref[0], part_ref[1]], axis=1)
    h = jax.nn.relu(dinv * agg + (dinv * dinv) * hw_ref[...] + b_ref[...])
    hw2 = jnp.dot(h, w2_ref[...], preferred_element_type=jnp.float32)
    u2_ref[0] = hw2 * dinv
    hw2_ref[...] = hw2


def _mid(part, hw1, cnt_col, b1p, w2p):
    return pl.pallas_call(
        _mid_body,
        grid=(N // BM, 2),
        in_specs=[
            pl.BlockSpec((2, BM, 128), lambda i, j: (0, i, 0)),
            pl.BlockSpec((BM, HP), lambda i, j: (i, 0)),
            pl.BlockSpec((BM, 1), lambda i, j: (i, 0)),
            pl.BlockSpec((1, HP), lambda i, j: (0, 0)),
            pl.BlockSpec((HP, 128), lambda i, j: (0, j)),
        ],
        out_specs=[
            pl.BlockSpec((1, BM, 128), lambda i, j: (j, i, 0)),
            pl.BlockSpec((BM, 128), lambda i, j: (i, j)),
        ],
        out_shape=[
            jax.ShapeDtypeStruct((2, N, 128), jnp.float32),
            jax.ShapeDtypeStruct((N, HP), jnp.float32),
        ],
    )(part, hw1, cnt_col, b1p, w2p)


def _final_body(part_ref, hw_ref, cnt_ref, b_ref, wl_ref, bl_ref, out_ref):
    i = pl.program_id(0)
    dinv = lax.rsqrt(jnp.maximum(cnt_ref[...] + 1.0, 1.0))
    agg = jnp.concatenate([part_ref[0], part_ref[1]], axis=1)
    h = jax.nn.relu(dinv * agg + (dinv * dinv) * hw_ref[...] + b_ref[...])
    drug = jnp.dot(h, wl_ref[...], preferred_element_type=jnp.float32)
    s = jnp.sum(drug, axis=0, keepdims=True) * (1.0 / N)

    @pl.when(i == 0)
    def _():
        out_ref[...] = bl_ref[...] + s

    @pl.when(i > 0)
    def _():
        out_ref[...] += s


def _final(part, hw2, cnt_col, b2p, wlp, bl_row):
    return pl.pallas_call(
        _final_body,
        grid=(N // BM,),
        in_specs=[
            pl.BlockSpec((2, BM, 128), lambda i: (0, i, 0)),
            pl.BlockSpec((BM, HP), lambda i: (i, 0)),
            pl.BlockSpec((BM, 1), lambda i: (i, 0)),
            pl.BlockSpec((1, HP), lambda i: (0, 0)),
            pl.BlockSpec((HP, 128), lambda i: (0, 0)),
            pl.BlockSpec((1, 128), lambda i: (0, 0)),
        ],
        out_specs=pl.BlockSpec((1, 128), lambda i: (0, 0)),
        out_shape=jax.ShapeDtypeStruct((1, 128), jnp.float32),
    )(part, hw2, cnt_col, b2p, wlp, bl_row)


def _sc_hist(dstidx, rowids, zrows):
    mesh = plsc.VectorSubcoreMesh(core_axis_name="c", subcore_axis_name="s")
    hchunk = NCHUNK // 2

    @functools.partial(
        pl.kernel,
        out_type=jax.ShapeDtypeStruct((2, QN, 128), jnp.float32),
        mesh=mesh,
        compiler_params=_sc_compiler_params(),
        scratch_types=[
            pltpu.VMEM((hchunk, CHUNK), jnp.int32),
            pltpu.VMEM((QN, 128), jnp.float32),
            pltpu.VMEM((1, QN), jnp.int32),
            pltpu.VMEM_SHARED((QN, 128), jnp.float32),
            pltpu.SemaphoreType.DMA,
        ],
    )
    def k(dst_hbm, row_hbm, z_hbm, out_hbm, dstv, lh, rids, sh, sem):
        c = lax.axis_index("c")
        s = lax.axis_index("s")
        ones = jnp.ones((16,), jnp.float32)

        @pl.loop(0, QN)
        def _(i):
            for kk in range(8):
                lh[i, pl.ds(16 * kk, 16)] = jnp.zeros((16,), jnp.float32)

        tile = c * 8 + s // 2
        half = s % 2
        pltpu.sync_copy(dst_hbm.at[tile, pl.ds(half * hchunk, hchunk)], dstv)
        pltpu.sync_copy(row_hbm, rids)

        @pl.when(s == 0)
        def _():
            pltpu.sync_copy(z_hbm.at[pl.ds(0, QN)], sh)

        @pl.loop(0, hchunk)
        def _(t):
            for kk in range(8):
                idx = dstv[t, pl.ds(16 * kk, 16)]
                plsc.addupdate_scatter(lh, [idx >> 7, idx & 127], ones)

        plsc.subcore_barrier()
        pltpu.sync_copy(lh, sh.at[rids.at[0]], add=True)
        plsc.subcore_barrier()

        @pl.when(s == 0)
        def _():
            pltpu.sync_copy(sh, out_hbm.at[c])

    return k(dstidx, rowids, zrows)


def _sc_agg(u2d, srcidx, dstidx, zrows):
    mesh = plsc.VectorSubcoreMesh(core_axis_name="c", subcore_axis_name="s")

    @functools.partial(
        pl.kernel,
        out_type=jax.ShapeDtypeStruct((2, N, 128), jnp.float32),
        mesh=mesh,
        compiler_params=_sc_compiler_params(),
        scratch_types=[
            pltpu.VMEM((GCHUNK, CHUNK), jnp.int32),
            pltpu.VMEM((GCHUNK, CHUNK), jnp.int32),
            pltpu.VMEM((CHUNK, 128), jnp.float32),
            pltpu.VMEM((CHUNK, 128), jnp.float32),
            pltpu.VMEM_SHARED((NROWS_PAD, 128), jnp.float32),
            pltpu.SemaphoreType.DMA,
            pltpu.SemaphoreType.DMA,
        ],
    )
    def k(u_hbm, src_hbm, dst_hbm, z_hbm, out_hbm,
          srcv, dstv, buf0, buf1, acc, sem0, sem1):
        c = lax.axis_index("c")
        s = lax.axis_index("s")
        pltpu.sync_copy(z_hbm, acc.at[pl.ds(s * ZROWS, ZROWS)])
        plsc.subcore_barrier()

        @pl.loop(0, NGRP)
        def _(g):
            pltpu.sync_copy(src_hbm.at[c, s, pl.ds(g * GCHUNK, GCHUNK)], srcv)
            pltpu.sync_copy(dst_hbm.at[s, pl.ds(g * GCHUNK, GCHUNK)], dstv)
            pltpu.async_copy(u_hbm.at[srcv.at[0]], buf0, sem0)

            @pl.loop(0, GCHUNK, step=2)
            def _(t):
                pltpu.async_copy(u_hbm.at[srcv.at[t + 1]], buf1, sem1)
                pltpu.make_async_copy(u_hbm.at[srcv.at[t]], buf0, sem0).wait()
                pltpu.sync_copy(buf0, acc.at[dstv.at[t]], add=True)

                @pl.when(t + 2 < GCHUNK)
                def _():
                    pltpu.async_copy(u_hbm.at[srcv.at[t + 2]], buf0, sem0)

                pltpu.make_async_copy(u_hbm.at[srcv.at[t + 1]], buf1, sem1).wait()
                pltpu.sync_copy(buf1, acc.at[dstv.at[t + 1]], add=True)

        plsc.subcore_barrier()
        pltpu.sync_copy(acc.at[pl.ds(s * OPT, OPT)],
                        out_hbm.at[c].at[pl.ds(s * OPT, OPT)])

        @pl.when(s == 0)
        def _():
            pltpu.sync_copy(acc.at[pl.ds(NT * OPT, N - NT * OPT)],
                            out_hbm.at[c].at[pl.ds(NT * OPT, N - NT * OPT)])

    return k(u2d, srcidx, dstidx, zrows)


def kernel(train_cll, x, edge_index, W1, b1, W2, b2, Wl, bl):
    f32 = jnp.float32
    src = edge_index[0]
    dst = edge_index[1]

    pad = EPT_PAD * NT - E
    src_p = jnp.concatenate([src, jnp.zeros((pad,), jnp.int32)])
    dst_p = jnp.concatenate([dst, jnp.full((pad,), N, jnp.int32)])
    src_t = src_p.reshape(NT, NCHUNK, CHUNK)
    srcidx = jnp.stack([src_t, src_t + N])
    dstidx = dst_p.reshape(NT, NCHUNK, CHUNK)
    zrows = jnp.zeros((ZROWS, 128), f32)
    rowids = jnp.arange(QN, dtype=jnp.int32).reshape(1, QN)

    w1p = jnp.zeros((128, HP), f32).at[:, :200].set(W1)
    w2p = jnp.zeros((HP, HP), f32).at[:200, :200].set(W2)
    wlp = jnp.zeros((HP, 128), f32).at[:200, :].set(Wl)
    b1p = jnp.zeros((1, HP), f32).at[0, :200].set(b1)
    b2p = jnp.zeros((1, HP), f32).at[0, :200].set(b2)
    bl_row = bl.reshape(1, 128)

    cnt2 = _sc_hist(dstidx, rowids, zrows)
    cnt_col = (cnt2[0] + cnt2[1]).reshape(QN * 128)[:N].reshape(N, 1)

    u1, hw1 = _prep(x, w1p, cnt_col)
    part1 = _sc_agg(u1.reshape(2 * N, 128), srcidx, dstidx, zrows)

    u2, hw2 = _mid(part1, hw1, cnt_col, b1p, w2p)
    part2 = _sc_agg(u2.reshape(2 * N, 128), srcidx, dstidx, zrows)

    pooled = _final(part2, hw2, cnt_col, b2p, wlp, bl_row)

    return jnp.concatenate([train_cll, pooled], axis=1)


# speedup vs baseline: 1.1813x; 1.1813x over previous
import dataclasses
import functools

import jax
import jax.numpy as jnp
from jax import lax
from jax.experimental import pallas as pl
from jax.experimental.pallas import tpu as pltpu
from jax.experimental.pallas import tpu_sc as plsc

N = 10000
E = 320000
HP = 256
QN = 80
BM = 1000
NT = 16
CHUNK = 128
NGRP = 4
GCHUNK = 40
NCHUNK = NGRP * GCHUNK
EPT_PAD = NCHUNK * CHUNK
ZROWS = 632
NROWS_PAD = NT * ZROWS
OPT = 624


def _sc_compiler_params():
    cp = pltpu.CompilerParams()
    if "needs_layout_passes" in pltpu.CompilerParams.__dataclass_fields__:
        cp = dataclasses.replace(cp, needs_layout_passes=False)
    return cp


def _prep_body(x_ref, w_ref, cnt_ref, u_ref, hw_ref):
    hw = jnp.dot(x_ref[...], w_ref[...], preferred_element_type=jnp.float32)
    dinv = lax.rsqrt(jnp.maximum(cnt_ref[...] + 1.0, 1.0))
    u_ref[0] = hw * dinv
    hw_ref[...] = hw


def _prep(x, w1p, cnt_col):
    return pl.pallas_call(
        _prep_body,
        grid=(N // BM, 2),
        in_specs=[
            pl.BlockSpec((BM, 128), lambda i, j: (i, 0)),
            pl.BlockSpec((128, 128), lambda i, j: (0, j)),
            pl.BlockSpec((BM, 1), lambda i, j: (i, 0)),
        ],
        out_specs=[
            pl.BlockSpec((1, BM, 128), lambda i, j: (j, i, 0)),
            pl.BlockSpec((BM, 128), lambda i, j: (i, j)),
        ],
        out_shape=[
            jax.ShapeDtypeStruct((2, N, 128), jnp.float32),
            jax.ShapeDtypeStruct((N, HP), jnp.float32),
        ],
    )(x, w1p, cnt_col)


def _mid_body(part_ref, hw_ref, cnt_ref, b_ref, w2_ref, u2_ref, hw2_ref):
    dinv = lax.rsqrt(jnp.maximum(cnt_ref[...] + 1.0, 1.0))
    agg = jnp.concatenate([part_ref[0], part_ref[1]], axis=1)
    h = jax.nn.relu(dinv * agg + (dinv * dinv) * hw_ref[...] + b_ref[...])
    hw2 = jnp.dot(h, w2_ref[...], preferred_element_type=jnp.float32)
    u2_ref[0] = hw2 * dinv
    hw2_ref[...] = hw2


def _mid(part, hw1, cnt_col, b1p, w2p):
    return pl.pallas_call(
        _mid_body,
        grid=(N // BM, 2),
        in_specs=[
            pl.BlockSpec((2, BM, 128), lambda i, j: (0, i, 0)),
            pl.BlockSpec((BM, HP), lambda i, j: (i, 0)),
            pl.BlockSpec((BM, 1), lambda i, j: (i, 0)),
            pl.BlockSpec((1, HP), lambda i, j: (0, 0)),
            pl.BlockSpec((HP, 128), lambda i, j: (0, j)),
        ],
        out_specs=[
            pl.BlockSpec((1, BM, 128), lambda i, j: (j, i, 0)),
            pl.BlockSpec((BM, 128), lambda i, j: (i, j)),
        ],
        out_shape=[
            jax.ShapeDtypeStruct((2, N, 128), jnp.float32),
            jax.ShapeDtypeStruct((N, HP), jnp.float32),
        ],
    )(part, hw1, cnt_col, b1p, w2p)


def _final_body(part_ref, hw_ref, cnt_ref, b_ref, wl_ref, bl_ref, out_ref):
    i = pl.program_id(0)
    dinv = lax.rsqrt(jnp.maximum(cnt_ref[...] + 1.0, 1.0))
    agg = jnp.concatenate([part_ref[0], part_ref[1]], axis=1)
    h = jax.nn.relu(dinv * agg + (dinv * dinv) * hw_ref[...] + b_ref[...])
    drug = jnp.dot(h, wl_ref[...], preferred_element_type=jnp.float32)
    s = jnp.sum(drug, axis=0, keepdims=True) * (1.0 / N)

    @pl.when(i == 0)
    def _():
        out_ref[...] = bl_ref[...] + s

    @pl.when(i > 0)
    def _():
        out_ref[...] += s


def _final(part, hw2, cnt_col, b2p, wlp, bl_row):
    return pl.pallas_call(
        _final_body,
        grid=(N // BM,),
        in_specs=[
            pl.BlockSpec((2, BM, 128), lambda i: (0, i, 0)),
            pl.BlockSpec((BM, HP), lambda i: (i, 0)),
            pl.BlockSpec((BM, 1), lambda i: (i, 0)),
            pl.BlockSpec((1, HP), lambda i: (0, 0)),
            pl.BlockSpec((HP, 128), lambda i: (0, 0)),
            pl.BlockSpec((1, 128), lambda i: (0, 0)),
        ],
        out_specs=pl.BlockSpec((1, 128), lambda i: (0, 0)),
        out_shape=jax.ShapeDtypeStruct((1, 128), jnp.float32),
    )(part, hw2, cnt_col, b2p, wlp, bl_row)


def _sc_hist(dstidx, rowids, zrows):
    mesh = plsc.VectorSubcoreMesh(core_axis_name="c", subcore_axis_name="s")
    hchunk = NCHUNK // 2

    @functools.partial(
        pl.kernel,
        out_type=jax.ShapeDtypeStruct((2, QN, 128), jnp.float32),
        mesh=mesh,
        compiler_params=_sc_compiler_params(),
        scratch_types=[
            pltpu.VMEM((hchunk, CHUNK), jnp.int32),
            pltpu.VMEM((QN, 128), jnp.float32),
            pltpu.VMEM((1, QN), jnp.int32),
            pltpu.VMEM_SHARED((QN, 128), jnp.float32),
            pltpu.SemaphoreType.DMA,
        ],
    )
    def k(dst_hbm, row_hbm, z_hbm, out_hbm, dstv, lh, rids, sh, sem):
        c = lax.axis_index("c")
        s = lax.axis_index("s")
        ones = jnp.ones((16,), jnp.float32)

        @pl.loop(0, QN)
        def _(i):
            for kk in range(8):
                lh[i, pl.ds(16 * kk, 16)] = jnp.zeros((16,), jnp.float32)

        tile = c * 8 + s // 2
        half = s % 2
        pltpu.sync_copy(dst_hbm.at[tile, pl.ds(half * hchunk, hchunk)], dstv)
        pltpu.sync_copy(row_hbm, rids)

        @pl.when(s == 0)
        def _():
            pltpu.sync_copy(z_hbm.at[pl.ds(0, QN)], sh)

        @pl.loop(0, hchunk)
        def _(t):
            for kk in range(8):
                idx = dstv[t, pl.ds(16 * kk, 16)]
                plsc.addupdate_scatter(lh, [idx >> 7, idx & 127], ones)

        plsc.subcore_barrier()
        pltpu.sync_copy(lh, sh.at[rids.at[0]], add=True)
        plsc.subcore_barrier()

        @pl.when(s == 0)
        def _():
            pltpu.sync_copy(sh, out_hbm.at[c])

    return k(dstidx, rowids, zrows)


def _sc_agg(u2d, srcidx, dstidx, zrows):
    mesh = plsc.VectorSubcoreMesh(core_axis_name="c", subcore_axis_name="s")

    @functools.partial(
        pl.kernel,
        out_type=jax.ShapeDtypeStruct((2, N, 128), jnp.float32),
        mesh=mesh,
        compiler_params=_sc_compiler_params(),
        scratch_types=[
            pltpu.VMEM((GCHUNK, CHUNK), jnp.int32),
            pltpu.VMEM((GCHUNK, CHUNK), jnp.int32),
            pltpu.VMEM((CHUNK, 128), jnp.float32),
            pltpu.VMEM((CHUNK, 128), jnp.float32),
            pltpu.VMEM_SHARED((NROWS_PAD, 128), jnp.float32),
            pltpu.SemaphoreType.DMA,
            pltpu.SemaphoreType.DMA,
        ],
    )
    def k(u_hbm, src_hbm, dst_hbm, z_hbm, out_hbm,
          srcv, dstv, buf0, buf1, acc, sem0, sem1):
        c = lax.axis_index("c")
        s = lax.axis_index("s")
        pltpu.sync_copy(z_hbm, acc.at[pl.ds(s * ZROWS, ZROWS)])
        coff = jnp.zeros((16,), jnp.int32) + c * N
        plsc.subcore_barrier()

        @pl.loop(0, NGRP)
        def _(g):
            pltpu.sync_copy(src_hbm.at[s, pl.ds(g * GCHUNK, GCHUNK)], srcv)
            pltpu.sync_copy(dst_hbm.at[s, pl.ds(g * GCHUNK, GCHUNK)], dstv)

            @pl.when(c == 1)
            def _():
                @pl.loop(0, GCHUNK)
                def _(t):
                    for kk in range(8):
                        srcv[t, pl.ds(16 * kk, 16)] += coff

            pltpu.async_copy(u_hbm.at[srcv.at[0]], buf0, sem0)

            @pl.loop(0, GCHUNK, step=2)
            def _(t):
                pltpu.async_copy(u_hbm.at[srcv.at[t + 1]], buf1, sem1)
                pltpu.make_async_copy(u_hbm.at[srcv.at[t]], buf0, sem0).wait()
                pltpu.sync_copy(buf0, acc.at[dstv.at[t]], add=True)

                @pl.when(t + 2 < GCHUNK)
                def _():
                    pltpu.async_copy(u_hbm.at[srcv.at[t + 2]], buf0, sem0)

                pltpu.make_async_copy(u_hbm.at[srcv.at[t + 1]], buf1, sem1).wait()
                pltpu.sync_copy(buf1, acc.at[dstv.at[t + 1]], add=True)

        plsc.subcore_barrier()
        pltpu.sync_copy(acc.at[pl.ds(s * OPT, OPT)],
                        out_hbm.at[c].at[pl.ds(s * OPT, OPT)])

        @pl.when(s == 0)
        def _():
            pltpu.sync_copy(acc.at[pl.ds(NT * OPT, N - NT * OPT)],
                            out_hbm.at[c].at[pl.ds(NT * OPT, N - NT * OPT)])

    return k(u2d, srcidx, dstidx, zrows)


def kernel(train_cll, x, edge_index, W1, b1, W2, b2, Wl, bl):
    f32 = jnp.float32
    src = edge_index[0]
    dst = edge_index[1]

    pad = EPT_PAD * NT - E
    src_p = jnp.concatenate([src, jnp.zeros((pad,), jnp.int32)])
    dst_p = jnp.concatenate([dst, jnp.full((pad,), N, jnp.int32)])
    srcidx = src_p.reshape(NT, NCHUNK, CHUNK)
    dstidx = dst_p.reshape(NT, NCHUNK, CHUNK)
    zrows = jnp.zeros((ZROWS, 128), f32)
    rowids = jnp.arange(QN, dtype=jnp.int32).reshape(1, QN)

    w1p = jnp.zeros((128, HP), f32).at[:, :200].set(W1)
    w2p = jnp.zeros((HP, HP), f32).at[:200, :200].set(W2)
    wlp = jnp.zeros((HP, 128), f32).at[:200, :].set(Wl)
    b1p = jnp.zeros((1, HP), f32).at[0, :200].set(b1)
    b2p = jnp.zeros((1, HP), f32).at[0, :200].set(b2)
    bl_row = bl.reshape(1, 128)

    cnt2 = _sc_hist(dstidx, rowids, zrows)
    cnt_col = (cnt2[0] + cnt2[1]).reshape(QN * 128)[:N].reshape(N, 1)

    u1, hw1 = _prep(x, w1p, cnt_col)
    part1 = _sc_agg(u1.reshape(2 * N, 128), srcidx, dstidx, zrows)

    u2, hw2 = _mid(part1, hw1, cnt_col, b1p, w2p)
    part2 = _sc_agg(u2.reshape(2 * N, 128), srcidx, dstidx, zrows)

    pooled = _final(part2, hw2, cnt_col, b2p, wlp, bl_row)

    return jnp.concatenate([train_cll, pooled], axis=1)


# speedup vs baseline: 1.1857x; 1.0037x over previous
import dataclasses
import functools

import jax
import jax.numpy as jnp
from jax import lax
from jax.experimental import pallas as pl
from jax.experimental.pallas import tpu as pltpu
from jax.experimental.pallas import tpu_sc as plsc

N = 10000
E = 320000
HP = 256
QN = 80
BM = 1000
NT = 16
CHUNK = 128
NGRP = 5
GCHUNK = 32
NCHUNK = NGRP * GCHUNK
EPT_PAD = NCHUNK * CHUNK
ZROWS = 632
NROWS_PAD = NT * ZROWS
OPT = 624


def _sc_compiler_params():
    cp = pltpu.CompilerParams()
    if "needs_layout_passes" in pltpu.CompilerParams.__dataclass_fields__:
        cp = dataclasses.replace(cp, needs_layout_passes=False)
    return cp


def _prep_body(x_ref, w_ref, cnt_ref, u_ref, hw_ref):
    hw = jnp.dot(x_ref[...], w_ref[...], preferred_element_type=jnp.float32)
    dinv = lax.rsqrt(jnp.maximum(cnt_ref[...] + 1.0, 1.0))
    u_ref[0] = hw * dinv
    hw_ref[...] = hw


def _prep(x, w1p, cnt_col):
    return pl.pallas_call(
        _prep_body,
        grid=(N // BM, 2),
        in_specs=[
            pl.BlockSpec((BM, 128), lambda i, j: (i, 0)),
            pl.BlockSpec((128, 128), lambda i, j: (0, j)),
            pl.BlockSpec((BM, 1), lambda i, j: (i, 0)),
        ],
        out_specs=[
            pl.BlockSpec((1, BM, 128), lambda i, j: (j, i, 0)),
            pl.BlockSpec((BM, 128), lambda i, j: (i, j)),
        ],
        out_shape=[
            jax.ShapeDtypeStruct((2, N, 128), jnp.float32),
            jax.ShapeDtypeStruct((N, HP), jnp.float32),
        ],
    )(x, w1p, cnt_col)


def _mid_body(part_ref, hw_ref, cnt_ref, b_ref, w2_ref, u2_ref, hw2_ref):
    dinv = lax.rsqrt(jnp.maximum(cnt_ref[...] + 1.0, 1.0))
    agg = jnp.concatenate([part_ref[0], part_ref[1]], axis=1)
    h = jax.nn.relu(dinv * agg + (dinv * dinv) * hw_ref[...] + b_ref[...])
    hw2 = jnp.dot(h, w2_ref[...], preferred_element_type=jnp.float32)
    u2_ref[0] = hw2 * dinv
    hw2_ref[...] = hw2


def _mid(part, hw1, cnt_col, b1p, w2p):
    return pl.pallas_call(
        _mid_body,
        grid=(N // BM, 2),
        in_specs=[
            pl.BlockSpec((2, BM, 128), lambda i, j: (0, i, 0)),
            pl.BlockSpec((BM, HP), lambda i, j: (i, 0)),
            pl.BlockSpec((BM, 1), lambda i, j: (i, 0)),
            pl.BlockSpec((1, HP), lambda i, j: (0, 0)),
            pl.BlockSpec((HP, 128), lambda i, j: (0, j)),
        ],
        out_specs=[
            pl.BlockSpec((1, BM, 128), lambda i, j: (j, i, 0)),
            pl.BlockSpec((BM, 128), lambda i, j: (i, j)),
        ],
        out_shape=[
            jax.ShapeDtypeStruct((2, N, 128), jnp.float32),
            jax.ShapeDtypeStruct((N, HP), jnp.float32),
        ],
    )(part, hw1, cnt_col, b1p, w2p)


def _final_body(part_ref, hw_ref, cnt_ref, b_ref, wl_ref, bl_ref, out_ref):
    i = pl.program_id(0)
    dinv = lax.rsqrt(jnp.maximum(cnt_ref[...] + 1.0, 1.0))
    agg = jnp.concatenate([part_ref[0], part_ref[1]], axis=1)
    h = jax.nn.relu(dinv * agg + (dinv * dinv) * hw_ref[...] + b_ref[...])
    drug = jnp.dot(h, wl_ref[...], preferred_element_type=jnp.float32)
    s = jnp.sum(drug, axis=0, keepdims=True) * (1.0 / N)

    @pl.when(i == 0)
    def _():
        out_ref[...] = bl_ref[...] + s

    @pl.when(i > 0)
    def _():
        out_ref[...] += s


def _final(part, hw2, cnt_col, b2p, wlp, bl_row):
    return pl.pallas_call(
        _final_body,
        grid=(N // BM,),
        in_specs=[
            pl.BlockSpec((2, BM, 128), lambda i: (0, i, 0)),
            pl.BlockSpec((BM, HP), lambda i: (i, 0)),
            pl.BlockSpec((BM, 1), lambda i: (i, 0)),
            pl.BlockSpec((1, HP), lambda i: (0, 0)),
            pl.BlockSpec((HP, 128), lambda i: (0, 0)),
            pl.BlockSpec((1, 128), lambda i: (0, 0)),
        ],
        out_specs=pl.BlockSpec((1, 128), lambda i: (0, 0)),
        out_shape=jax.ShapeDtypeStruct((1, 128), jnp.float32),
    )(part, hw2, cnt_col, b2p, wlp, bl_row)


def _sc_hist(dstidx, rowids, zrows):
    mesh = plsc.VectorSubcoreMesh(core_axis_name="c", subcore_axis_name="s")
    hchunk = NCHUNK // 2

    @functools.partial(
        pl.kernel,
        out_type=jax.ShapeDtypeStruct((2, QN, 128), jnp.float32),
        mesh=mesh,
        compiler_params=_sc_compiler_params(),
        scratch_types=[
            pltpu.VMEM((hchunk, CHUNK), jnp.int32),
            pltpu.VMEM((QN, 128), jnp.float32),
            pltpu.VMEM((1, QN), jnp.int32),
            pltpu.VMEM_SHARED((QN, 128), jnp.float32),
            pltpu.SemaphoreType.DMA,
        ],
    )
    def k(dst_hbm, row_hbm, z_hbm, out_hbm, dstv, lh, rids, sh, sem):
        c = lax.axis_index("c")
        s = lax.axis_index("s")
        ones = jnp.ones((16,), jnp.float32)

        @pl.loop(0, QN)
        def _(i):
            for kk in range(8):
                lh[i, pl.ds(16 * kk, 16)] = jnp.zeros((16,), jnp.float32)

        tile = c * 8 + s // 2
        half = s % 2
        pltpu.sync_copy(dst_hbm.at[tile, pl.ds(half * hchunk, hchunk)], dstv)
        pltpu.sync_copy(row_hbm, rids)

        @pl.when(s == 0)
        def _():
            pltpu.sync_copy(z_hbm.at[pl.ds(0, QN)], sh)

        @pl.loop(0, hchunk)
        def _(t):
            for kk in range(8):
                idx = dstv[t, pl.ds(16 * kk, 16)]
                plsc.addupdate_scatter(lh, [idx >> 7, idx & 127], ones)

        plsc.subcore_barrier()
        pltpu.sync_copy(lh, sh.at[rids.at[0]], add=True)
        plsc.subcore_barrier()

        @pl.when(s == 0)
        def _():
            pltpu.sync_copy(sh, out_hbm.at[c])

    return k(dstidx, rowids, zrows)


def _sc_agg(u2d, srcidx, dstidx, zrows):
    mesh = plsc.VectorSubcoreMesh(core_axis_name="c", subcore_axis_name="s")

    @functools.partial(
        pl.kernel,
        out_type=jax.ShapeDtypeStruct((2, N, 128), jnp.float32),
        mesh=mesh,
        compiler_params=_sc_compiler_params(),
        scratch_types=[
            pltpu.VMEM((GCHUNK, CHUNK), jnp.int32),
            pltpu.VMEM((GCHUNK, CHUNK), jnp.int32),
            pltpu.VMEM((GCHUNK, CHUNK), jnp.int32),
            pltpu.VMEM((GCHUNK, CHUNK), jnp.int32),
            pltpu.VMEM((CHUNK, 128), jnp.float32),
            pltpu.VMEM((CHUNK, 128), jnp.float32),
            pltpu.VMEM_SHARED((NROWS_PAD, 128), jnp.float32),
            pltpu.SemaphoreType.DMA,
            pltpu.SemaphoreType.DMA,
            pltpu.SemaphoreType.DMA,
        ],
    )
    def k(u_hbm, src_hbm, dst_hbm, z_hbm, out_hbm,
          srcv0, dstv0, srcv1, dstv1, buf0, buf1, acc, sem0, sem1, isem):
        c = lax.axis_index("c")
        s = lax.axis_index("s")
        pltpu.sync_copy(z_hbm, acc.at[pl.ds(s * ZROWS, ZROWS)])
        coff = jnp.zeros((16,), jnp.int32) + c * N
        plsc.subcore_barrier()

        sv = [srcv0, srcv1]
        dv = [dstv0, dstv1]
        pltpu.sync_copy(src_hbm.at[s, pl.ds(0, GCHUNK)], srcv0)
        pltpu.sync_copy(dst_hbm.at[s, pl.ds(0, GCHUNK)], dstv0)

        for g in range(NGRP):
            srcv = sv[g % 2]
            dstv = dv[g % 2]
            if g + 1 < NGRP:
                pltpu.async_copy(
                    src_hbm.at[s, pl.ds((g + 1) * GCHUNK, GCHUNK)],
                    sv[(g + 1) % 2], isem)
                pltpu.async_copy(
                    dst_hbm.at[s, pl.ds((g + 1) * GCHUNK, GCHUNK)],
                    dv[(g + 1) % 2], isem)

            @pl.when(c == 1)
            def _():
                @pl.loop(0, GCHUNK)
                def _(t):
                    for kk in range(8):
                        srcv[t, pl.ds(16 * kk, 16)] += coff

            pltpu.async_copy(u_hbm.at[srcv.at[0]], buf0, sem0)

            @pl.loop(0, GCHUNK, step=2)
            def _(t):
                pltpu.async_copy(u_hbm.at[srcv.at[t + 1]], buf1, sem1)
                pltpu.make_async_copy(u_hbm.at[srcv.at[t]], buf0, sem0).wait()
                pltpu.sync_copy(buf0, acc.at[dstv.at[t]], add=True)

                @pl.when(t + 2 < GCHUNK)
                def _():
                    pltpu.async_copy(u_hbm.at[srcv.at[t + 2]], buf0, sem0)

                pltpu.make_async_copy(u_hbm.at[srcv.at[t + 1]], buf1, sem1).wait()
                pltpu.sync_copy(buf1, acc.at[dstv.at[t + 1]], add=True)

            if g + 1 < NGRP:
                pltpu.make_async_copy(
                    src_hbm.at[s, pl.ds((g + 1) * GCHUNK, GCHUNK)],
                    sv[(g + 1) % 2], isem).wait()
                pltpu.make_async_copy(
                    dst_hbm.at[s, pl.ds((g + 1) * GCHUNK, GCHUNK)],
                    dv[(g + 1) % 2], isem).wait()

        plsc.subcore_barrier()
        pltpu.sync_copy(acc.at[pl.ds(s * OPT, OPT)],
                        out_hbm.at[c].at[pl.ds(s * OPT, OPT)])

        @pl.when(s == 0)
        def _():
            pltpu.sync_copy(acc.at[pl.ds(NT * OPT, N - NT * OPT)],
                            out_hbm.at[c].at[pl.ds(NT * OPT, N - NT * OPT)])

    return k(u2d, srcidx, dstidx, zrows)


def kernel(train_cll, x, edge_index, W1, b1, W2, b2, Wl, bl):
    f32 = jnp.float32
    src = edge_index[0]
    dst = edge_index[1]

    pad = EPT_PAD * NT - E
    src_p = jnp.concatenate([src, jnp.zeros((pad,), jnp.int32)])
    dst_p = jnp.concatenate([dst, jnp.full((pad,), N, jnp.int32)])
    srcidx = src_p.reshape(NT, NCHUNK, CHUNK)
    dstidx = dst_p.reshape(NT, NCHUNK, CHUNK)
    zrows = jnp.zeros((ZROWS, 128), f32)
    rowids = jnp.arange(QN, dtype=jnp.int32).reshape(1, QN)

    w1p = jnp.zeros((128, HP), f32).at[:, :200].set(W1)
    w2p = jnp.zeros((HP, HP), f32).at[:200, :200].set(W2)
    wlp = jnp.zeros((HP, 128), f32).at[:200, :].set(Wl)
    b1p = jnp.zeros((1, HP), f32).at[0, :200].set(b1)
    b2p = jnp.zeros((1, HP), f32).at[0, :200].set(b2)
    bl_row = bl.reshape(1, 128)

    cnt2 = _sc_hist(dstidx, rowids, zrows)
    cnt_col = (cnt2[0] + cnt2[1]).reshape(QN * 128)[:N].reshape(N, 1)

    u1, hw1 = _prep(x, w1p, cnt_col)
    part1 = _sc_agg(u1.reshape(2 * N, 128), srcidx, dstidx, zrows)

    u2, hw2 = _mid(part1, hw1, cnt_col, b1p, w2p)
    part2 = _sc_agg(u2.reshape(2 * N, 128), srcidx, dstidx, zrows)

    pooled = _final(part2, hw2, cnt_col, b2p, wlp, bl_row)

    return jnp.concatenate([train_cll, pooled], axis=1)


# speedup vs baseline: 1.1872x; 1.0013x over previous
import dataclasses
import functools

import jax
import jax.numpy as jnp
from jax import lax
from jax.experimental import pallas as pl
from jax.experimental.pallas import tpu as pltpu
from jax.experimental.pallas import tpu_sc as plsc

N = 10000
E = 320000
HP = 256
QN = 80
BM = 1000
NT = 16
CHUNK = 128
NGRP = 5
GCHUNK = 32
NCHUNK = NGRP * GCHUNK
EPT_PAD = NCHUNK * CHUNK
ZROWS = 632
NROWS_PAD = NT * ZROWS
OPT = 624


def _sc_compiler_params():
    cp = pltpu.CompilerParams()
    if "needs_layout_passes" in pltpu.CompilerParams.__dataclass_fields__:
        cp = dataclasses.replace(cp, needs_layout_passes=False)
    return cp


def _prep_body(x_ref, w_ref, cnt_ref, u_ref, hw_ref):
    hw = jnp.dot(x_ref[...], w_ref[...], preferred_element_type=jnp.float32)
    dinv = lax.rsqrt(jnp.maximum(cnt_ref[...] + 1.0, 1.0))
    u_ref[0] = hw * dinv
    hw_ref[...] = hw


def _prep(x, w1p, cnt_col):
    return pl.pallas_call(
        _prep_body,
        grid=(N // BM, 2),
        in_specs=[
            pl.BlockSpec((BM, 128), lambda i, j: (i, 0)),
            pl.BlockSpec((128, 128), lambda i, j: (0, j)),
            pl.BlockSpec((BM, 1), lambda i, j: (i, 0)),
        ],
        out_specs=[
            pl.BlockSpec((1, BM, 128), lambda i, j: (j, i, 0)),
            pl.BlockSpec((BM, 128), lambda i, j: (i, j)),
        ],
        out_shape=[
            jax.ShapeDtypeStruct((2, N, 128), jnp.float32),
            jax.ShapeDtypeStruct((N, HP), jnp.float32),
        ],
    )(x, w1p, cnt_col)


def _mid_body(part_ref, hw_ref, cnt_ref, b_ref, w2_ref, u2_ref, hw2_ref):
    dinv = lax.rsqrt(jnp.maximum(cnt_ref[...] + 1.0, 1.0))
    agg = jnp.concatenate([part_ref[0], part_ref[1]], axis=1)
    h = jax.nn.relu(dinv * agg + (dinv * dinv) * hw_ref[...] + b_ref[...])
    hw2 = jnp.dot(h, w2_ref[...], preferred_element_type=jnp.float32)
    u2_ref[0] = hw2 * dinv
    hw2_ref[...] = hw2


def _mid(part, hw1, cnt_col, b1p, w2p):
    return pl.pallas_call(
        _mid_body,
        grid=(N // BM, 2),
        in_specs=[
            pl.BlockSpec((2, BM, 128), lambda i, j: (0, i, 0)),
            pl.BlockSpec((BM, HP), lambda i, j: (i, 0)),
            pl.BlockSpec((BM, 1), lambda i, j: (i, 0)),
            pl.BlockSpec((1, HP), lambda i, j: (0, 0)),
            pl.BlockSpec((HP, 128), lambda i, j: (0, j)),
        ],
        out_specs=[
            pl.BlockSpec((1, BM, 128), lambda i, j: (j, i, 0)),
            pl.BlockSpec((BM, 128), lambda i, j: (i, j)),
        ],
        out_shape=[
            jax.ShapeDtypeStruct((2, N, 128), jnp.float32),
            jax.ShapeDtypeStruct((N, HP), jnp.float32),
        ],
    )(part, hw1, cnt_col, b1p, w2p)


def _final_body(part_ref, hw_ref, cnt_ref, b_ref, wl_ref, bl_ref, out_ref):
    i = pl.program_id(0)
    dinv = lax.rsqrt(jnp.maximum(cnt_ref[...] + 1.0, 1.0))
    agg = jnp.concatenate([part_ref[0], part_ref[1]], axis=1)
    h = jax.nn.relu(dinv * agg + (dinv * dinv) * hw_ref[...] + b_ref[...])
    drug = jnp.dot(h, wl_ref[...], preferred_element_type=jnp.float32)
    s = jnp.sum(drug, axis=0, keepdims=True) * (1.0 / N)

    @pl.when(i == 0)
    def _():
        out_ref[...] = bl_ref[...] + s

    @pl.when(i > 0)
    def _():
        out_ref[...] += s


def _final(part, hw2, cnt_col, b2p, wlp, bl_row):
    return pl.pallas_call(
        _final_body,
        grid=(N // BM,),
        in_specs=[
            pl.BlockSpec((2, BM, 128), lambda i: (0, i, 0)),
            pl.BlockSpec((BM, HP), lambda i: (i, 0)),
            pl.BlockSpec((BM, 1), lambda i: (i, 0)),
            pl.BlockSpec((1, HP), lambda i: (0, 0)),
            pl.BlockSpec((HP, 128), lambda i: (0, 0)),
            pl.BlockSpec((1, 128), lambda i: (0, 0)),
        ],
        out_specs=pl.BlockSpec((1, 128), lambda i: (0, 0)),
        out_shape=jax.ShapeDtypeStruct((1, 128), jnp.float32),
    )(part, hw2, cnt_col, b2p, wlp, bl_row)


def _sc_hist(dstidx, rowids, zrows):
    mesh = plsc.VectorSubcoreMesh(core_axis_name="c", subcore_axis_name="s")
    hchunk = NCHUNK // 2

    @functools.partial(
        pl.kernel,
        out_type=jax.ShapeDtypeStruct((2, QN, 128), jnp.float32),
        mesh=mesh,
        compiler_params=_sc_compiler_params(),
        scratch_types=[
            pltpu.VMEM((hchunk, CHUNK), jnp.int32),
            pltpu.VMEM((QN, 128), jnp.float32),
            pltpu.VMEM((1, QN), jnp.int32),
            pltpu.VMEM_SHARED((QN, 128), jnp.float32),
            pltpu.SemaphoreType.DMA,
        ],
    )
    def k(dst_hbm, row_hbm, z_hbm, out_hbm, dstv, lh, rids, sh, sem):
        c = lax.axis_index("c")
        s = lax.axis_index("s")
        ones = jnp.ones((16,), jnp.float32)

        @pl.loop(0, QN)
        def _(i):
            for kk in range(8):
                lh[i, pl.ds(16 * kk, 16)] = jnp.zeros((16,), jnp.float32)

        tile = c * 8 + s // 2
        half = s % 2
        pltpu.sync_copy(dst_hbm.at[tile, pl.ds(half * hchunk, hchunk)], dstv)
        pltpu.sync_copy(row_hbm, rids)

        @pl.when(s == 0)
        def _():
            pltpu.sync_copy(z_hbm.at[pl.ds(0, QN)], sh)

        @pl.loop(0, hchunk)
        def _(t):
            for kk in range(8):
                idx = dstv[t, pl.ds(16 * kk, 16)]
                plsc.addupdate_scatter(lh, [idx >> 7, idx & 127], ones)

        plsc.subcore_barrier()
        pltpu.sync_copy(lh, sh.at[rids.at[0]], add=True)
        plsc.subcore_barrier()

        @pl.when(s == 0)
        def _():
            pltpu.sync_copy(sh, out_hbm.at[c])

    return k(dstidx, rowids, zrows)


def _sc_agg(u2d, srcidx, dstidx, zrows):
    mesh = plsc.VectorSubcoreMesh(core_axis_name="c", subcore_axis_name="s")

    @functools.partial(
        pl.kernel,
        out_type=jax.ShapeDtypeStruct((2, N, 128), jnp.float32),
        mesh=mesh,
        compiler_params=_sc_compiler_params(),
        scratch_types=[
            pltpu.VMEM((GCHUNK, CHUNK), jnp.int32),
            pltpu.VMEM((GCHUNK, CHUNK), jnp.int32),
            pltpu.VMEM((GCHUNK, CHUNK), jnp.int32),
            pltpu.VMEM((GCHUNK, CHUNK), jnp.int32),
            pltpu.VMEM((CHUNK, 128), jnp.float32),
            pltpu.VMEM((CHUNK, 128), jnp.float32),
            pltpu.VMEM_SHARED((NROWS_PAD, 128), jnp.float32),
            pltpu.SemaphoreType.DMA,
            pltpu.SemaphoreType.DMA,
            pltpu.SemaphoreType.DMA,
        ],
    )
    def k(u_hbm, src_hbm, dst_hbm, z_hbm, out_hbm,
          srcv0, dstv0, srcv1, dstv1, buf0, buf1, acc, sem0, sem1, isem):
        c = lax.axis_index("c")
        s = lax.axis_index("s")
        pltpu.sync_copy(z_hbm, acc.at[pl.ds(s * ZROWS, ZROWS)])
        plsc.subcore_barrier()
        uc = u_hbm.at[c]

        sv = [srcv0, srcv1]
        dv = [dstv0, dstv1]
        pltpu.sync_copy(src_hbm.at[s, pl.ds(0, GCHUNK)], srcv0)
        pltpu.sync_copy(dst_hbm.at[s, pl.ds(0, GCHUNK)], dstv0)

        for g in range(NGRP):
            srcv = sv[g % 2]
            dstv = dv[g % 2]
            if g + 1 < NGRP:
                pltpu.async_copy(
                    src_hbm.at[s, pl.ds((g + 1) * GCHUNK, GCHUNK)],
                    sv[(g + 1) % 2], isem)
                pltpu.async_copy(
                    dst_hbm.at[s, pl.ds((g + 1) * GCHUNK, GCHUNK)],
                    dv[(g + 1) % 2], isem)

            pltpu.async_copy(uc.at[srcv.at[0]], buf0, sem0)

            @pl.loop(0, GCHUNK, step=2)
            def _(t):
                pltpu.async_copy(uc.at[srcv.at[t + 1]], buf1, sem1)
                pltpu.make_async_copy(uc.at[srcv.at[t]], buf0, sem0).wait()
                pltpu.sync_copy(buf0, acc.at[dstv.at[t]], add=True)

                @pl.when(t + 2 < GCHUNK)
                def _():
                    pltpu.async_copy(uc.at[srcv.at[t + 2]], buf0, sem0)

                pltpu.make_async_copy(uc.at[srcv.at[t + 1]], buf1, sem1).wait()
                pltpu.sync_copy(buf1, acc.at[dstv.at[t + 1]], add=True)

            if g + 1 < NGRP:
                pltpu.make_async_copy(
                    src_hbm.at[s, pl.ds((g + 1) * GCHUNK, GCHUNK)],
                    sv[(g + 1) % 2], isem).wait()
                pltpu.make_async_copy(
                    dst_hbm.at[s, pl.ds((g + 1) * GCHUNK, GCHUNK)],
                    dv[(g + 1) % 2], isem).wait()

        plsc.subcore_barrier()
        pltpu.sync_copy(acc.at[pl.ds(s * OPT, OPT)],
                        out_hbm.at[c].at[pl.ds(s * OPT, OPT)])

        @pl.when(s == 0)
        def _():
            pltpu.sync_copy(acc.at[pl.ds(NT * OPT, N - NT * OPT)],
                            out_hbm.at[c].at[pl.ds(NT * OPT, N - NT * OPT)])

    return k(u2d, srcidx, dstidx, zrows)


def kernel(train_cll, x, edge_index, W1, b1, W2, b2, Wl, bl):
    f32 = jnp.float32
    src = edge_index[0]
    dst = edge_index[1]

    pad = EPT_PAD * NT - E
    src_p = jnp.concatenate([src, jnp.zeros((pad,), jnp.int32)])
    dst_p = jnp.concatenate([dst, jnp.full((pad,), N, jnp.int32)])
    srcidx = src_p.reshape(NT, NCHUNK, CHUNK)
    dstidx = dst_p.reshape(NT, NCHUNK, CHUNK)
    zrows = jnp.zeros((ZROWS, 128), f32)
    rowids = jnp.arange(QN, dtype=jnp.int32).reshape(1, QN)

    w1p = jnp.zeros((128, HP), f32).at[:, :200].set(W1)
    w2p = jnp.zeros((HP, HP), f32).at[:200, :200].set(W2)
    wlp = jnp.zeros((HP, 128), f32).at[:200, :].set(Wl)
    b1p = jnp.zeros((1, HP), f32).at[0, :200].set(b1)
    b2p = jnp.zeros((1, HP), f32).at[0, :200].set(b2)
    bl_row = bl.reshape(1, 128)

    cnt2 = _sc_hist(dstidx, rowids, zrows)
    cnt_col = (cnt2[0] + cnt2[1]).reshape(QN * 128)[:N].reshape(N, 1)

    u1, hw1 = _prep(x, w1p, cnt_col)
    part1 = _sc_agg(u1, srcidx, dstidx, zrows)

    u2, hw2 = _mid(part1, hw1, cnt_col, b1p, w2p)
    part2 = _sc_agg(u2, srcidx, dstidx, zrows)

    pooled = _final(part2, hw2, cnt_col, b2p, wlp, bl_row)

    return jnp.concatenate([train_cll, pooled], axis=1)
